# Initial kernel scaffold; baseline (speedup 1.0000x reference)
import jax
import jax.numpy as jnp
from jax.experimental import pallas as pl


def kernel(ids, wte, wpe):
    raise NotImplementedError("write your pallas kernel here")


# speedup vs baseline: 1.1586x; 1.1586x over previous
import functools

import jax
import jax.numpy as jnp
from jax import lax
from jax.experimental import pallas as pl
from jax.experimental.pallas import tpu as pltpu
from jax.experimental.pallas import tpu_sc as plsc

B = 4
T = 2048
D = 768
L = 16
NVEC = D // L

_info = plsc.get_sparse_core_info()
NC, NS = _info.num_cores, _info.num_subcores
NW = NC * NS
TPW = T // NW


def _body(ids_hbm, wte_hbm, wpe_hbm, out_hbm, idx_v, tok_v, pos_v, sem):
    wid = lax.axis_index("s") * NC + lax.axis_index("c")
    t0 = wid * TPW

    pltpu.sync_copy(wpe_hbm.at[pl.ds(t0, TPW)], pos_v)

    for b in range(B):
        base = b * T + t0
        pltpu.sync_copy(ids_hbm.at[pl.ds(base, TPW)], idx_v)
        pltpu.async_copy(wte_hbm.at[idx_v], tok_v, sem).wait()

        def add_row(i, carry):
            for j in range(NVEC):
                sl = pl.ds(j * L, L)
                tok_v[i, sl] = tok_v[i, sl] + pos_v[i, sl]
            return carry

        lax.fori_loop(0, TPW, add_row, 0)
        pltpu.sync_copy(tok_v, out_hbm.at[pl.ds(base, TPW)])


@jax.jit
def kernel(ids, wte, wpe):
    mesh = plsc.VectorSubcoreMesh(core_axis_name="c", subcore_axis_name="s")
    run = functools.partial(
        pl.kernel,
        mesh=mesh,
        out_type=jax.ShapeDtypeStruct((B * T, D), jnp.float32),
        scratch_types=[
            pltpu.VMEM((TPW,), jnp.int32),
            pltpu.VMEM((TPW, D), jnp.float32),
            pltpu.VMEM((TPW, D), jnp.float32),
            pltpu.SemaphoreType.DMA,
        ],
    )(_body)
    out = run(ids.reshape(B * T).astype(jnp.int32), wte, wpe)
    return out.reshape(B, T, D)
